# baseline (speedup 1.0000x reference)
import functools

import jax
import jax.numpy as jnp
from jax import lax
from jax.experimental import pallas as pl
from jax.experimental.pallas import tpu as pltpu
from jax.experimental.pallas import tpu_sc as plsc

N_REG = 10000
N_SUBJ = 2048
H = 256
HALF = 128
OUT = 64
NC = 2
NS = 16
CHUNK = 128

N_ACC_S = 2176
N_ACC_R = 10112
RPS_R = N_ACC_R // NS

RBLK = 2000
NB_R = N_REG // RBLK


def _pad_edges(src, dst, garbage, mult):
    e = src.shape[0]
    ep = ((e + mult - 1) // mult) * mult
    pad = ep - e
    srcp = jnp.concatenate([src.astype(jnp.int32), jnp.zeros((pad,), jnp.int32)])
    dstp = jnp.concatenate([dst.astype(jnp.int32),
                            jnp.full((pad,), garbage, jnp.int32)])
    return srcp, dstp


_MESH = plsc.VectorSubcoreMesh(core_axis_name="c", subcore_axis_name="s")


def _make_counts(e_s_pad, e_r_pad):
    eps_s = e_s_pad // NS
    eps_r = e_r_pad // NS

    @functools.partial(
        pl.kernel,
        out_type=(jax.ShapeDtypeStruct((N_ACC_S, 16), jnp.float32),
                  jax.ShapeDtypeStruct((N_ACC_R, 16), jnp.float32)),
        mesh=_MESH,
        scratch_types=[
            pltpu.VMEM((CHUNK,), jnp.int32),
            pltpu.VMEM((CHUNK, 16), jnp.float32),
            pltpu.VMEM_SHARED((N_ACC_R, 16), jnp.float32),
            pltpu.SemaphoreType.DMA,
        ],
    )
    def k(dst_s, dst_r, ones_hbm, zeros16_hbm, cs_out, cr_out,
          idx_v, ones_v, acc, sem):
        cid = lax.axis_index("c")
        sid = lax.axis_index("s")
        pltpu.sync_copy(ones_hbm, ones_v)

        def stage(dst_hbm, n_acc, eps, out_ref):
            rps = n_acc // NS
            pltpu.sync_copy(zeros16_hbm.at[pl.ds(0, rps)],
                            acc.at[pl.ds(sid * rps, rps)])
            plsc.subcore_barrier()

            def body(b, carry):
                base = sid * eps + b * CHUNK
                pltpu.sync_copy(dst_hbm.at[pl.ds(base, CHUNK)], idx_v)
                pltpu.sync_copy(ones_v, acc.at[idx_v], add=True)
                return carry

            lax.fori_loop(0, eps // CHUNK, body, 0)
            plsc.subcore_barrier()
            pltpu.sync_copy(acc.at[pl.ds(sid * rps, rps)],
                            out_ref.at[pl.ds(sid * rps, rps)])

        @pl.when(cid == 0)
        def _():
            stage(dst_s, N_ACC_S, eps_s, cs_out)

        @pl.when(cid == 1)
        def _():
            stage(dst_r, N_ACC_R, eps_r, cr_out)

    return k


IB = 8
NBUF = 2
DEPTH = 1


def _make_gather_add(e_s_pad, e_r_pad):
    nblk_sub_s = e_s_pad // (NS * CHUNK)
    nblk_sub_r = e_r_pad // (NS * CHUNK)
    assert nblk_sub_s % IB == 0 and nblk_sub_r % IB == 0

    @functools.partial(
        pl.kernel,
        out_type=(jax.ShapeDtypeStruct((N_ACC_S, H), jnp.float32),
                  jax.ShapeDtypeStruct((N_ACC_R, H), jnp.float32)),
        mesh=_MESH,
        scratch_types=[
            pltpu.VMEM((IB, CHUNK), jnp.int32),
            pltpu.VMEM((IB, CHUNK), jnp.int32),
            pltpu.VMEM((NBUF, CHUNK, HALF), jnp.float32),
            pltpu.VMEM_SHARED((N_ACC_R, HALF), jnp.float32),
        ] + [pltpu.SemaphoreType.DMA] * (2 * NBUF),
    )
    def k(hrA, hrB, tA, tB, src_s, dst_s, src_r, dst_r, zeros_hbm,
          s_out, r_out, src_v, dst_v, rows_v, acc, *sems):
        sem_g = sems[:NBUF]
        sem_s = sems[NBUF:]
        cid = lax.axis_index("c")
        sid = lax.axis_index("s")

        def stage(tab, src2d, dst2d, n_acc, nblk_sub, out_ref, c0):
            rps = n_acc // NS
            pltpu.sync_copy(zeros_hbm.at[pl.ds(0, rps)],
                            acc.at[pl.ds(sid * rps, rps)])
            plsc.subcore_barrier()
            blk0 = sid * nblk_sub

            def batch(bat, carry):
                row0 = blk0 + bat * IB
                pltpu.sync_copy(src2d.at[pl.ds(row0, IB)], src_v)
                pltpu.sync_copy(dst2d.at[pl.ds(row0, IB)], dst_v)
                g = {}
                s = {}

                def start_scatter(k_):
                    b_ = k_ % NBUF
                    g[k_].wait()
                    s[k_] = pltpu.async_copy(
                        rows_v.at[b_], acc.at[dst_v.at[k_]], sem_s[b_],
                        add=True)

                for kk in range(IB):
                    b = kk % NBUF
                    if kk >= NBUF:
                        s[kk - NBUF].wait()
                    g[kk] = pltpu.async_copy(
                        tab.at[src_v.at[kk]], rows_v.at[b], sem_g[b])
                    if kk >= DEPTH:
                        start_scatter(kk - DEPTH)
                for kk in range(IB - DEPTH, IB):
                    start_scatter(kk)
                for kk in range(IB - NBUF, IB):
                    s[kk].wait()
                return carry

            lax.fori_loop(0, nblk_sub // IB, batch, 0)
            plsc.subcore_barrier()
            pltpu.sync_copy(
                acc.at[pl.ds(sid * rps, rps)],
                out_ref.at[pl.ds(sid * rps, rps), pl.ds(c0, HALF)])
            plsc.subcore_barrier()

        @pl.when(cid == 0)
        def _():
            stage(hrA, src_s, dst_s, N_ACC_S, nblk_sub_s, s_out, 0)
            stage(tA, src_r, dst_r, N_ACC_R, nblk_sub_r, r_out, 0)

        @pl.when(cid == 1)
        def _():
            stage(hrB, src_s, dst_s, N_ACC_S, nblk_sub_s, s_out, HALF)
            stage(tB, src_r, dst_r, N_ACC_R, nblk_sub_r, r_out, HALF)

    return k


_PREC = jax.lax.Precision.HIGHEST


def _dot(a, b):
    return jnp.dot(a, b, precision=_PREC, preferred_element_type=jnp.float32)


def _f32(shape):
    return jax.ShapeDtypeStruct(shape, jnp.float32)


def _prep_s_body(xs, lsw, lsb, cs_part, cr_part, hs_out, cinv_out, dinv_out):
    x = xs[...]
    w = lsw[...]
    hs_out[...] = (x[:, 0:1] * w[0:1, :] + x[:, 1:2] * w[1:2, :]
                   + x[:, 2:3] * w[2:3, :] + lsb[...])
    cinv_out[...] = 1.0 / jnp.maximum(cs_part[0:N_SUBJ, 0:1], 1.0)
    dinv_out[...] = lax.rsqrt(cr_part[0:N_REG, 0:1] + 1.0)


def _prep_r_body(xr, lrw, lrb, dinv, hrA_out, hrB_out, tA_out, tB_out):
    hr = _dot(xr[...], lrw[...]) + lrb[...]
    t = dinv[...] * hr
    hrA_out[...] = hr[:, :HALF]
    hrB_out[...] = hr[:, HALF:]
    tA_out[...] = t[:, :HALF]
    tB_out[...] = t[:, HALF:]


def _bn_relu_from_stats(x, stats, n, gamma, beta):
    mu = stats[0:1, :] / n
    var = stats[1:2, :] / n - mu * mu
    return jnp.maximum((x - mu) * lax.rsqrt(var + 1e-5) * gamma + beta, 0.0)


def _comb_s_body(s_part, hs_prev, cinv, wl, wr, bs, gamma, beta, hs_out):
    agg = s_part[0:N_SUBJ, :] * cinv[...]
    new_s = _dot(agg, wl[...]) + _dot(hs_prev[...], wr[...]) + bs[...]
    mu = jnp.mean(new_s, axis=0, keepdims=True)
    var = jnp.mean((new_s - mu) * (new_s - mu), axis=0, keepdims=True)
    hs_out[...] = jnp.maximum(
        (new_s - mu) * lax.rsqrt(var + 1e-5) * gamma[...] + beta[...], 0.0)


def _comb_s_last_body(s_part, hs_prev, cinv, wl, wr, bs, gamma, beta,
                      osw, osb, outs_out):
    agg = s_part[0:N_SUBJ, :] * cinv[...]
    new_s = _dot(agg, wl[...]) + _dot(hs_prev[...], wr[...]) + bs[...]
    mu = jnp.mean(new_s, axis=0, keepdims=True)
    var = jnp.mean((new_s - mu) * (new_s - mu), axis=0, keepdims=True)
    hs = jnp.maximum(
        (new_s - mu) * lax.rsqrt(var + 1e-5) * gamma[...] + beta[...], 0.0)
    outs_out[...] = _dot(hs, osw[...]) + osb[...]


def _r_phase_a_body(r_part, tA, tB, dinv, gw, bg, newr_out, stats_out,
                    stats_acc):
    i = pl.program_id(0)

    @pl.when(i == 0)
    def _():
        stats_acc[...] = jnp.zeros((8, H), jnp.float32)

    t = jnp.concatenate([tA[...], tB[...]], axis=1)
    m = dinv[...] * (r_part[...] + t)
    new_r = _dot(m, gw[...]) + bg[...]
    newr_out[...] = new_r
    psum = jnp.sum(new_r, axis=0, keepdims=True)
    psq = jnp.sum(new_r * new_r, axis=0, keepdims=True)
    stats_acc[0:1, :] += psum
    stats_acc[1:2, :] += psq
    stats_out[...] = stats_acc[...]


def _r_phase_b_body(newr, stats, dinv, gamma, beta, hrA_out, hrB_out,
                    tA_out, tB_out):
    hr = _bn_relu_from_stats(newr[...], stats[...], float(N_REG),
                             gamma[...], beta[...])
    t = dinv[...] * hr
    hrA_out[...] = hr[:, :HALF]
    hrB_out[...] = hr[:, HALF:]
    tA_out[...] = t[:, :HALF]
    tB_out[...] = t[:, HALF:]


def _r_phase_b_last_body(newr, stats, gamma, beta, orw, orb, outr_out):
    hr = _bn_relu_from_stats(newr[...], stats[...], float(N_REG),
                             gamma[...], beta[...])
    outr_out[...] = _dot(hr, orw[...]) + orb[...]


def _rows(shape_rows, cols):
    return pl.BlockSpec((shape_rows, cols), lambda i: (i, 0))


def _whole(r, c):
    return pl.BlockSpec((r, c), lambda i: (0, 0))


def kernel(x_subject, x_region, edge_index_rs, edge_index_rr, lin_s_w,
           lin_s_b, lin_r_w, lin_r_b, sage_wl, sage_wr, sage_b, gcn_w,
           gcn_b, bn_gamma, bn_beta, out_s_w, out_s_b, out_r_w, out_r_b):
    mult = NS * CHUNK * IB
    src_s, dst_s = _pad_edges(edge_index_rs[0], edge_index_rs[1], N_SUBJ, mult)
    src_r, dst_r = _pad_edges(edge_index_rr[0], edge_index_rr[1], N_REG, mult)
    e_s_pad = src_s.shape[0]
    e_r_pad = src_r.shape[0]
    src_s2 = src_s.reshape(-1, CHUNK)
    dst_s2 = dst_s.reshape(-1, CHUNK)
    src_r2 = src_r.reshape(-1, CHUNK)
    dst_r2 = dst_r.reshape(-1, CHUNK)

    ones16 = jnp.ones((CHUNK, 16), jnp.float32)
    zeros16 = jnp.zeros((RPS_R, 16), jnp.float32)
    zeros128 = jnp.zeros((RPS_R, HALF), jnp.float32)

    cs_part, cr_part = _make_counts(e_s_pad, e_r_pad)(
        dst_s, dst_r, ones16, zeros16)

    hs, cinv, dinv = pl.pallas_call(
        _prep_s_body,
        out_shape=(_f32((N_SUBJ, H)), _f32((N_SUBJ, 1)), _f32((N_REG, 1))),
    )(x_subject, lin_s_w, lin_s_b.reshape(1, H), cs_part, cr_part)

    hrA, hrB, tA, tB = pl.pallas_call(
        _prep_r_body,
        grid=(NB_R,),
        in_specs=[_rows(RBLK, H), _whole(H, H), _whole(1, H), _rows(RBLK, 1)],
        out_specs=(_rows(RBLK, HALF),) * 4,
        out_shape=(_f32((N_REG, HALF)),) * 4,
    )(x_region, lin_r_w, lin_r_b.reshape(1, H), dinv)

    gath = _make_gather_add(e_s_pad, e_r_pad)

    for layer in range(2):
        last = layer == 1
        s_sum, r_sum = gath(hrA, hrB, tA, tB, src_s2, dst_s2, src_r2,
                            dst_r2, zeros128)
        bs = sage_b[layer].reshape(1, H)
        bg = gcn_b[layer].reshape(1, H)
        gamma = bn_gamma[layer].reshape(1, H)
        beta = bn_beta[layer].reshape(1, H)

        if not last:
            hs = pl.pallas_call(
                _comb_s_body, out_shape=_f32((N_SUBJ, H)),
            )(s_sum, hs, cinv, sage_wl[layer], sage_wr[layer], bs, gamma, beta)
        else:
            out_s = pl.pallas_call(
                _comb_s_last_body, out_shape=_f32((N_SUBJ, OUT)),
            )(s_sum, hs, cinv, sage_wl[layer], sage_wr[layer], bs, gamma,
              beta, out_s_w, out_s_b.reshape(1, OUT))

        new_r, stats = pl.pallas_call(
            _r_phase_a_body,
            grid=(NB_R,),
            in_specs=[_rows(RBLK, H), _rows(RBLK, HALF), _rows(RBLK, HALF),
                      _rows(RBLK, 1), _whole(H, H), _whole(1, H)],
            out_specs=(_rows(RBLK, H), _whole(8, H)),
            out_shape=(_f32((N_REG, H)), _f32((8, H))),
            scratch_shapes=[pltpu.VMEM((8, H), jnp.float32)],
        )(r_sum, tA, tB, dinv, gcn_w[layer], bg)

        if not last:
            hrA, hrB, tA, tB = pl.pallas_call(
                _r_phase_b_body,
                grid=(NB_R,),
                in_specs=[_rows(RBLK, H), _whole(8, H), _rows(RBLK, 1),
                          _whole(1, H), _whole(1, H)],
                out_specs=(_rows(RBLK, HALF),) * 4,
                out_shape=(_f32((N_REG, HALF)),) * 4,
            )(new_r, stats, dinv, gamma, beta)
        else:
            out_r = pl.pallas_call(
                _r_phase_b_last_body,
                grid=(NB_R,),
                in_specs=[_rows(RBLK, H), _whole(8, H), _whole(1, H),
                          _whole(1, H), _whole(H, OUT), _whole(1, OUT)],
                out_specs=_rows(RBLK, OUT),
                out_shape=_f32((N_REG, OUT)),
            )(new_r, stats, gamma, beta, out_r_w, out_r_b.reshape(1, OUT))

    return (out_s, out_r)


# speedup vs baseline: 1.9903x; 1.9903x over previous
import functools

import jax
import jax.numpy as jnp
from jax import lax
from jax.experimental import pallas as pl
from jax.experimental.pallas import tpu as pltpu
from jax.experimental.pallas import tpu_sc as plsc

N_REG = 10000
N_SUBJ = 2048
H = 256
HALF = 128
OUT = 64
NC = 2
NS = 16
CHUNK = 128

N_ACC_S = 2176
N_ACC_R = 10112
RPS_R = N_ACC_R // NS

RBLK = 2000
NB_R = N_REG // RBLK


def _pad_edges(src, dst, garbage, mult):
    e = src.shape[0]
    ep = ((e + mult - 1) // mult) * mult
    pad = ep - e
    srcp = jnp.concatenate([src.astype(jnp.int32), jnp.zeros((pad,), jnp.int32)])
    dstp = jnp.concatenate([dst.astype(jnp.int32),
                            jnp.full((pad,), garbage, jnp.int32)])
    return srcp, dstp


_MESH = plsc.VectorSubcoreMesh(core_axis_name="c", subcore_axis_name="s")


def _make_counts(e_s_pad, e_r_pad):
    eps_s = e_s_pad // NS
    eps_r = e_r_pad // NS

    @functools.partial(
        pl.kernel,
        out_type=(jax.ShapeDtypeStruct((N_ACC_S, 16), jnp.float32),
                  jax.ShapeDtypeStruct((N_ACC_R, 16), jnp.float32)),
        mesh=_MESH,
        scratch_types=[
            pltpu.VMEM((CHUNK,), jnp.int32),
            pltpu.VMEM((CHUNK, 16), jnp.float32),
            pltpu.VMEM_SHARED((N_ACC_R, 16), jnp.float32),
            pltpu.SemaphoreType.DMA,
        ],
    )
    def k(dst_s, dst_r, ones_hbm, zeros16_hbm, cs_out, cr_out,
          idx_v, ones_v, acc, sem):
        cid = lax.axis_index("c")
        sid = lax.axis_index("s")
        pltpu.sync_copy(ones_hbm, ones_v)

        def stage(dst_hbm, n_acc, eps, out_ref):
            rps = n_acc // NS
            pltpu.sync_copy(zeros16_hbm.at[pl.ds(0, rps)],
                            acc.at[pl.ds(sid * rps, rps)])
            plsc.subcore_barrier()

            def body(b, carry):
                base = sid * eps + b * CHUNK
                pltpu.sync_copy(dst_hbm.at[pl.ds(base, CHUNK)], idx_v)
                pltpu.sync_copy(ones_v, acc.at[idx_v], add=True)
                return carry

            lax.fori_loop(0, eps // CHUNK, body, 0)
            plsc.subcore_barrier()
            pltpu.sync_copy(acc.at[pl.ds(sid * rps, rps)],
                            out_ref.at[pl.ds(sid * rps, rps)])

        @pl.when(cid == 0)
        def _():
            stage(dst_s, N_ACC_S, eps_s, cs_out)

        @pl.when(cid == 1)
        def _():
            stage(dst_r, N_ACC_R, eps_r, cr_out)

    return k


def _make_gather_add(e_s_pad, e_r_pad):
    nblk_sub_s = e_s_pad // (NS * CHUNK)
    nblk_sub_r = e_r_pad // (NS * CHUNK)
    assert nblk_sub_s % 2 == 0 and nblk_sub_r % 2 == 0

    @functools.partial(
        pl.kernel,
        out_type=(jax.ShapeDtypeStruct((N_ACC_S, H), jnp.float32),
                  jax.ShapeDtypeStruct((N_ACC_R, H), jnp.float32)),
        mesh=_MESH,
        scratch_types=[
            pltpu.VMEM((CHUNK,), jnp.int32),
            pltpu.VMEM((CHUNK,), jnp.int32),
            pltpu.VMEM((CHUNK,), jnp.int32),
            pltpu.VMEM((CHUNK,), jnp.int32),
            pltpu.VMEM((CHUNK, HALF), jnp.float32),
            pltpu.VMEM((CHUNK, HALF), jnp.float32),
            pltpu.VMEM_SHARED((N_ACC_R, HALF), jnp.float32),
            pltpu.SemaphoreType.DMA,
            pltpu.SemaphoreType.DMA,
        ],
    )
    def k(hrA, hrB, tA, tB, src_s, dst_s, src_r, dst_r, zeros_hbm,
          s_out, r_out, src_v0, src_v1, dst_v0, dst_v1, buf0, buf1,
          acc, sem0, sem1):
        cid = lax.axis_index("c")
        sid = lax.axis_index("s")

        def stage(tab, src_hbm, dst_hbm, n_acc, nblk_sub, out_ref, c0):
            rps = n_acc // NS
            pltpu.sync_copy(zeros_hbm.at[pl.ds(0, rps)],
                            acc.at[pl.ds(sid * rps, rps)])
            plsc.subcore_barrier()
            base0 = sid * nblk_sub * CHUNK

            def gath_start(blk, src_v, buf, sem):
                pltpu.sync_copy(src_hbm.at[pl.ds(base0 + blk * CHUNK, CHUNK)],
                                src_v)
                pltpu.async_copy(tab.at[src_v], buf, sem)

            def gath_wait(src_v, buf, sem):
                pltpu.make_async_copy(tab.at[src_v], buf, sem).wait()

            def scat(blk, dst_v, buf):
                pltpu.sync_copy(dst_hbm.at[pl.ds(base0 + blk * CHUNK, CHUNK)],
                                dst_v)
                pltpu.sync_copy(buf, acc.at[dst_v], add=True)

            def pair(i, issue_next):
                e = 2 * i
                gath_start(e + 1, src_v1, buf1, sem1)
                gath_wait(src_v0, buf0, sem0)
                scat(e, dst_v0, buf0)
                if issue_next:
                    gath_start(e + 2, src_v0, buf0, sem0)
                gath_wait(src_v1, buf1, sem1)
                scat(e + 1, dst_v1, buf1)

            npair = nblk_sub // 2
            gath_start(0, src_v0, buf0, sem0)

            def body(i, carry):
                pair(i, True)
                return carry

            lax.fori_loop(0, npair - 1, body, 0)
            pair(npair - 1, False)
            plsc.subcore_barrier()
            pltpu.sync_copy(
                acc.at[pl.ds(sid * rps, rps)],
                out_ref.at[pl.ds(sid * rps, rps), pl.ds(c0, HALF)])
            plsc.subcore_barrier()

        @pl.when(cid == 0)
        def _():
            stage(hrA, src_s, dst_s, N_ACC_S, nblk_sub_s, s_out, 0)
            stage(tA, src_r, dst_r, N_ACC_R, nblk_sub_r, r_out, 0)

        @pl.when(cid == 1)
        def _():
            stage(hrB, src_s, dst_s, N_ACC_S, nblk_sub_s, s_out, HALF)
            stage(tB, src_r, dst_r, N_ACC_R, nblk_sub_r, r_out, HALF)

    return k


_PREC = jax.lax.Precision.HIGHEST


def _dot(a, b):
    return jnp.dot(a, b, precision=_PREC, preferred_element_type=jnp.float32)


def _f32(shape):
    return jax.ShapeDtypeStruct(shape, jnp.float32)


def _prep_s_body(xs, lsw, lsb, cs_part, cr_part, hs_out, cinv_out, dinv_out):
    x = xs[...]
    w = lsw[...]
    hs_out[...] = (x[:, 0:1] * w[0:1, :] + x[:, 1:2] * w[1:2, :]
                   + x[:, 2:3] * w[2:3, :] + lsb[...])
    cinv_out[...] = 1.0 / jnp.maximum(cs_part[0:N_SUBJ, 0:1], 1.0)
    dinv_out[...] = lax.rsqrt(cr_part[0:N_REG, 0:1] + 1.0)


def _prep_r_body(xr, lrw, lrb, dinv, hrA_out, hrB_out, tA_out, tB_out):
    hr = _dot(xr[...], lrw[...]) + lrb[...]
    t = dinv[...] * hr
    hrA_out[...] = hr[:, :HALF]
    hrB_out[...] = hr[:, HALF:]
    tA_out[...] = t[:, :HALF]
    tB_out[...] = t[:, HALF:]


def _bn_relu_from_stats(x, stats, n, gamma, beta):
    mu = stats[0:1, :] / n
    var = stats[1:2, :] / n - mu * mu
    return jnp.maximum((x - mu) * lax.rsqrt(var + 1e-5) * gamma + beta, 0.0)


def _comb_s_body(s_part, hs_prev, cinv, wl, wr, bs, gamma, beta, hs_out):
    agg = s_part[0:N_SUBJ, :] * cinv[...]
    new_s = _dot(agg, wl[...]) + _dot(hs_prev[...], wr[...]) + bs[...]
    mu = jnp.mean(new_s, axis=0, keepdims=True)
    var = jnp.mean((new_s - mu) * (new_s - mu), axis=0, keepdims=True)
    hs_out[...] = jnp.maximum(
        (new_s - mu) * lax.rsqrt(var + 1e-5) * gamma[...] + beta[...], 0.0)


def _comb_s_last_body(s_part, hs_prev, cinv, wl, wr, bs, gamma, beta,
                      osw, osb, outs_out):
    agg = s_part[0:N_SUBJ, :] * cinv[...]
    new_s = _dot(agg, wl[...]) + _dot(hs_prev[...], wr[...]) + bs[...]
    mu = jnp.mean(new_s, axis=0, keepdims=True)
    var = jnp.mean((new_s - mu) * (new_s - mu), axis=0, keepdims=True)
    hs = jnp.maximum(
        (new_s - mu) * lax.rsqrt(var + 1e-5) * gamma[...] + beta[...], 0.0)
    outs_out[...] = _dot(hs, osw[...]) + osb[...]


def _r_phase_a_body(r_part, tA, tB, dinv, gw, bg, newr_out, stats_out,
                    stats_acc):
    i = pl.program_id(0)

    @pl.when(i == 0)
    def _():
        stats_acc[...] = jnp.zeros((8, H), jnp.float32)

    t = jnp.concatenate([tA[...], tB[...]], axis=1)
    m = dinv[...] * (r_part[...] + t)
    new_r = _dot(m, gw[...]) + bg[...]
    newr_out[...] = new_r
    psum = jnp.sum(new_r, axis=0, keepdims=True)
    psq = jnp.sum(new_r * new_r, axis=0, keepdims=True)
    stats_acc[0:1, :] += psum
    stats_acc[1:2, :] += psq
    stats_out[...] = stats_acc[...]


def _r_phase_b_body(newr, stats, dinv, gamma, beta, hrA_out, hrB_out,
                    tA_out, tB_out):
    hr = _bn_relu_from_stats(newr[...], stats[...], float(N_REG),
                             gamma[...], beta[...])
    t = dinv[...] * hr
    hrA_out[...] = hr[:, :HALF]
    hrB_out[...] = hr[:, HALF:]
    tA_out[...] = t[:, :HALF]
    tB_out[...] = t[:, HALF:]


def _r_phase_b_last_body(newr, stats, gamma, beta, orw, orb, outr_out):
    hr = _bn_relu_from_stats(newr[...], stats[...], float(N_REG),
                             gamma[...], beta[...])
    outr_out[...] = _dot(hr, orw[...]) + orb[...]


def _rows(shape_rows, cols):
    return pl.BlockSpec((shape_rows, cols), lambda i: (i, 0))


def _whole(r, c):
    return pl.BlockSpec((r, c), lambda i: (0, 0))


def kernel(x_subject, x_region, edge_index_rs, edge_index_rr, lin_s_w,
           lin_s_b, lin_r_w, lin_r_b, sage_wl, sage_wr, sage_b, gcn_w,
           gcn_b, bn_gamma, bn_beta, out_s_w, out_s_b, out_r_w, out_r_b):
    mult = NS * CHUNK * 2
    src_s, dst_s = _pad_edges(edge_index_rs[0], edge_index_rs[1], N_SUBJ, mult)
    src_r, dst_r = _pad_edges(edge_index_rr[0], edge_index_rr[1], N_REG, mult)
    e_s_pad = src_s.shape[0]
    e_r_pad = src_r.shape[0]

    ones16 = jnp.ones((CHUNK, 16), jnp.float32)
    zeros16 = jnp.zeros((RPS_R, 16), jnp.float32)
    zeros128 = jnp.zeros((RPS_R, HALF), jnp.float32)

    cs_part, cr_part = _make_counts(e_s_pad, e_r_pad)(
        dst_s, dst_r, ones16, zeros16)

    hs, cinv, dinv = pl.pallas_call(
        _prep_s_body,
        out_shape=(_f32((N_SUBJ, H)), _f32((N_SUBJ, 1)), _f32((N_REG, 1))),
    )(x_subject, lin_s_w, lin_s_b.reshape(1, H), cs_part, cr_part)

    hrA, hrB, tA, tB = pl.pallas_call(
        _prep_r_body,
        grid=(NB_R,),
        in_specs=[_rows(RBLK, H), _whole(H, H), _whole(1, H), _rows(RBLK, 1)],
        out_specs=(_rows(RBLK, HALF),) * 4,
        out_shape=(_f32((N_REG, HALF)),) * 4,
    )(x_region, lin_r_w, lin_r_b.reshape(1, H), dinv)

    gath = _make_gather_add(e_s_pad, e_r_pad)

    for layer in range(2):
        last = layer == 1
        s_sum, r_sum = gath(hrA, hrB, tA, tB, src_s, dst_s, src_r, dst_r,
                            zeros128)
        bs = sage_b[layer].reshape(1, H)
        bg = gcn_b[layer].reshape(1, H)
        gamma = bn_gamma[layer].reshape(1, H)
        beta = bn_beta[layer].reshape(1, H)

        if not last:
            hs = pl.pallas_call(
                _comb_s_body, out_shape=_f32((N_SUBJ, H)),
            )(s_sum, hs, cinv, sage_wl[layer], sage_wr[layer], bs, gamma, beta)
        else:
            out_s = pl.pallas_call(
                _comb_s_last_body, out_shape=_f32((N_SUBJ, OUT)),
            )(s_sum, hs, cinv, sage_wl[layer], sage_wr[layer], bs, gamma,
              beta, out_s_w, out_s_b.reshape(1, OUT))

        new_r, stats = pl.pallas_call(
            _r_phase_a_body,
            grid=(NB_R,),
            in_specs=[_rows(RBLK, H), _rows(RBLK, HALF), _rows(RBLK, HALF),
                      _rows(RBLK, 1), _whole(H, H), _whole(1, H)],
            out_specs=(_rows(RBLK, H), _whole(8, H)),
            out_shape=(_f32((N_REG, H)), _f32((8, H))),
            scratch_shapes=[pltpu.VMEM((8, H), jnp.float32)],
        )(r_sum, tA, tB, dinv, gcn_w[layer], bg)

        if not last:
            hrA, hrB, tA, tB = pl.pallas_call(
                _r_phase_b_body,
                grid=(NB_R,),
                in_specs=[_rows(RBLK, H), _whole(8, H), _rows(RBLK, 1),
                          _whole(1, H), _whole(1, H)],
                out_specs=(_rows(RBLK, HALF),) * 4,
                out_shape=(_f32((N_REG, HALF)),) * 4,
            )(new_r, stats, dinv, gamma, beta)
        else:
            out_r = pl.pallas_call(
                _r_phase_b_last_body,
                grid=(NB_R,),
                in_specs=[_rows(RBLK, H), _whole(8, H), _whole(1, H),
                          _whole(1, H), _whole(H, OUT), _whole(1, OUT)],
                out_specs=_rows(RBLK, OUT),
                out_shape=_f32((N_REG, OUT)),
            )(new_r, stats, gamma, beta, out_r_w, out_r_b.reshape(1, OUT))

    return (out_s, out_r)
